# baseline (speedup 1.0000x reference)
import functools

import jax
import jax.numpy as jnp
from jax import lax
from jax.experimental import pallas as pl
from jax.experimental.pallas import tpu as pltpu
from jax.experimental.pallas import tpu_sc as plsc

_NC = 2
_NS = 16
_NW = _NC * _NS
_L = 16

_MESH = dict(core_axis_name="c", subcore_axis_name="s", num_cores=_NC,
             num_subcores=_NS)


def _proj_body(x_ref, w1_ref, b1_ref, a_ref, b_ref):
    d = x_ref.shape[1]
    x = x_ref[...]
    w1a = w1_ref[:d, :]
    w1b = w1_ref[d:, :]
    a_ref[...] = (
        jnp.dot(x, w1a - w1b, preferred_element_type=jnp.float32) + b1_ref[...]
    )
    b_ref[...] = jnp.dot(x, w1b, preferred_element_type=jnp.float32)


def _node_proj(x, W1, b1):
    n, d = x.shape
    bn = 1000
    return pl.pallas_call(
        _proj_body,
        grid=(n // bn,),
        in_specs=[
            pl.BlockSpec((bn, d), lambda i: (i, 0)),
            pl.BlockSpec((2 * d, d), lambda i: (0, 0)),
            pl.BlockSpec((1, d), lambda i: (0, 0)),
        ],
        out_specs=[
            pl.BlockSpec((bn, d), lambda i: (i, 0)),
            pl.BlockSpec((bn, d), lambda i: (i, 0)),
        ],
        out_shape=[
            jax.ShapeDtypeStruct((n, d), jnp.float32),
            jax.ShapeDtypeStruct((n, d), jnp.float32),
        ],
    )(x, W1, b1.reshape(1, d))


def _edge_gather(A, B, src, dst):
    n, d = A.shape
    e = src.shape[0]
    c2 = 80
    nchunk = e // c2
    cpt = -(-nchunk // _NW)
    npair = -(-cpt // 2)

    def body(a_hbm, b_hbm, src_hbm, dst_hbm, g_hbm,
             idxd0, idxs0, bufa0, bufb0, idxd1, idxs1, bufa1, bufb1,
             semA0, semB0, semO0, semA1, semB1, semO1):
        wid = lax.axis_index("s") * _NC + lax.axis_index("c")
        slots = ((idxd0, idxs0, bufa0, bufb0, semA0, semB0, semO0),
                 (idxd1, idxs1, bufa1, bufb1, semA1, semB1, semO1))

        def pair_body(bi, carry):
            c0 = (2 * bi) * _NW + wid
            cs = (c0, c0 + _NW)
            for s in range(2):
                idxd, idxs, bufa, bufb, semA, semB, semO = slots[s]

                @pl.when(cs[s] < nchunk)
                def _():
                    base = cs[s] * c2
                    pltpu.sync_copy(dst_hbm.at[pl.ds(base, c2)], idxd)
                    pltpu.sync_copy(src_hbm.at[pl.ds(base, c2)], idxs)
                    pltpu.async_copy(a_hbm.at[idxd], bufa, semA)
                    pltpu.async_copy(b_hbm.at[idxs], bufb, semB)

            for s in range(2):
                idxd, idxs, bufa, bufb, semA, semB, semO = slots[s]

                @pl.when(cs[s] < nchunk)
                def _():
                    pltpu.make_async_copy(a_hbm.at[idxd], bufa, semA).wait()
                    pltpu.make_async_copy(b_hbm.at[idxs], bufb, semB).wait()

                    def add_row(r, carry2):
                        for k in range(d // _L):
                            bufa[r, pl.ds(k * _L, _L)] = (
                                bufa[r, pl.ds(k * _L, _L)]
                                + bufb[r, pl.ds(k * _L, _L)])
                        return carry2

                    lax.fori_loop(0, c2, add_row, 0)
                    base = cs[s] * c2
                    pltpu.async_copy(bufa, g_hbm.at[pl.ds(base, c2)], semO)

            for s in range(2):
                idxd, idxs, bufa, bufb, semA, semB, semO = slots[s]

                @pl.when(cs[s] < nchunk)
                def _():
                    base = cs[s] * c2
                    pltpu.make_async_copy(
                        bufa, g_hbm.at[pl.ds(base, c2)], semO).wait()

            return carry

        lax.fori_loop(0, npair, pair_body, 0)

    f = pl.kernel(
        body,
        out_type=jax.ShapeDtypeStruct((e, d), jnp.float32),
        mesh=plsc.VectorSubcoreMesh(**_MESH),
        scratch_types=[
            pltpu.VMEM((c2,), jnp.int32),
            pltpu.VMEM((c2,), jnp.int32),
            pltpu.VMEM((c2, d), jnp.float32),
            pltpu.VMEM((c2, d), jnp.float32),
            pltpu.VMEM((c2,), jnp.int32),
            pltpu.VMEM((c2,), jnp.int32),
            pltpu.VMEM((c2, d), jnp.float32),
            pltpu.VMEM((c2, d), jnp.float32),
            pltpu.SemaphoreType.DMA,
            pltpu.SemaphoreType.DMA,
            pltpu.SemaphoreType.DMA,
            pltpu.SemaphoreType.DMA,
            pltpu.SemaphoreType.DMA,
            pltpu.SemaphoreType.DMA,
        ],
        compiler_params=pltpu.CompilerParams(needs_layout_passes=False),
    )
    return f(A, B, src, dst)


def _mlp_body(g_ref, w2_ref, b2_ref, h_ref):
    g = jnp.maximum(g_ref[...], 0.0)
    h_ref[...] = (
        jnp.dot(g, w2_ref[...], preferred_element_type=jnp.float32) + b2_ref[...]
    )


def _edge_mlp(G, W2, b2):
    e, d = G.shape
    be = 640
    return pl.pallas_call(
        _mlp_body,
        grid=(e // be,),
        in_specs=[
            pl.BlockSpec((be, d), lambda i: (i, 0)),
            pl.BlockSpec((d, d), lambda i: (0, 0)),
            pl.BlockSpec((1, d), lambda i: (0, 0)),
        ],
        out_specs=pl.BlockSpec((be, d), lambda i: (i, 0)),
        out_shape=jax.ShapeDtypeStruct((e, d), jnp.float32),
    )(G, W2, b2.reshape(1, d))


def _segment_max(dst, H, n):
    e, d = H.shape
    npt = -(-n // _NW)
    npad = npt * _NW
    scan = 2000
    nscan = e // scan
    gc = 64
    accw = npt * d

    assert nscan % 2 == 0
    npairs = nscan // 2

    def body(dst_hbm, h_hbm, out_hbm, acc,
             dbufA, ebufA, lbufA, hbufA, dbufB, ebufB, lbufB, hbufB,
             semDA, semHA, semDB, semHB):
        wid = lax.axis_index("s") * _NC + lax.axis_index("c")
        lo = wid * npt
        neg_inf = jnp.full((_L,), -jnp.inf, jnp.float32)
        iota = lax.iota(jnp.int32, _L)

        def init_body(i, carry):
            acc[pl.ds(i * _L, _L)] = neg_inf
            return carry

        lax.fori_loop(0, accw // _L, init_body, 0)

        def scan_chunk(sc, dbuf, ebuf, lbuf):
            def filt_body(k, cnt):
                dv = dbuf[pl.ds(k * _L, _L)]
                m = (dv >= lo) & (dv < lo + npt)
                mi = jnp.where(m, 1, 0)
                csum = plsc.cumsum(mi)
                pos = cnt + csum - mi
                eid = sc * scan + k * _L + iota
                plsc.store_scatter(ebuf, [pos], eid, mask=m)
                plsc.store_scatter(lbuf, [pos], dv - lo, mask=m)
                return cnt + plsc.all_reduce_population_count(m)

            cnt0 = jnp.zeros((_L,), jnp.int32)
            cvec = lax.fori_loop(0, scan // _L, filt_body, cnt0)
            zeros = jnp.zeros((_L,), jnp.int32)
            for t in range(gc // _L):
                plsc.store_scatter(ebuf, [cvec + t * _L + iota], zeros)
            return cvec[0]

        def accum_group(g, count, ebuf, lbuf, hbuf):
            nrows = jnp.minimum(gc, count - g * gc)
            nb = (nrows + _L - 1) // _L

            def blk_body(q, carry2):
                dlv = lbuf[pl.ds(g * gc + q * _L, _L)]
                rbase = q * _L
                for j in range(_L):
                    @pl.when(rbase + j < nrows)
                    def _(j=j):
                        rb = dlv[j] * d
                        for k in range(d // _L):
                            off = rb + k * _L
                            acc[pl.ds(off, _L)] = jnp.maximum(
                                acc[pl.ds(off, _L)],
                                hbuf[rbase + j, pl.ds(k * _L, _L)])
                return carry2

            lax.fori_loop(0, nb, blk_body, 0)

        def extra_groups(count, ebuf, lbuf, hbuf, semH):
            ngroups = (count + gc - 1) // gc

            def g_body(g, carry2):
                pltpu.async_copy(h_hbm.at[ebuf.at[pl.ds(g * gc, gc)]],
                                 hbuf, semH).wait()
                accum_group(g, count, ebuf, lbuf, hbuf)
                return carry2

            lax.fori_loop(1, ngroups, g_body, 0)

        pltpu.async_copy(dst_hbm.at[pl.ds(0, scan)], dbufA, semDA)

        def pair_body(bi, carry):
            sc0 = 2 * bi
            sc1 = sc0 + 1
            pltpu.make_async_copy(
                dst_hbm.at[pl.ds(sc0 * scan, scan)], dbufA, semDA).wait()
            pltpu.async_copy(dst_hbm.at[pl.ds(sc1 * scan, scan)], dbufB, semDB)
            count0 = scan_chunk(sc0, dbufA, ebufA, lbufA)
            pltpu.async_copy(h_hbm.at[ebufA.at[pl.ds(0, gc)]], hbufA, semHA)

            @pl.when(bi + 1 < npairs)
            def _():
                pltpu.async_copy(
                    dst_hbm.at[pl.ds((sc0 + 2) * scan, scan)], dbufA, semDA)

            pltpu.make_async_copy(
                dst_hbm.at[pl.ds(sc1 * scan, scan)], dbufB, semDB).wait()
            count1 = scan_chunk(sc1, dbufB, ebufB, lbufB)
            pltpu.async_copy(h_hbm.at[ebufB.at[pl.ds(0, gc)]], hbufB, semHB)

            pltpu.make_async_copy(
                h_hbm.at[ebufA.at[pl.ds(0, gc)]], hbufA, semHA).wait()
            accum_group(0, count0, ebufA, lbufA, hbufA)
            extra_groups(count0, ebufA, lbufA, hbufA, semHA)

            pltpu.make_async_copy(
                h_hbm.at[ebufB.at[pl.ds(0, gc)]], hbufB, semHB).wait()
            accum_group(0, count1, ebufB, lbufB, hbufB)
            extra_groups(count1, ebufB, lbufB, hbufB, semHB)
            return carry

        lax.fori_loop(0, npairs, pair_body, 0)

        def fix_body(i, carry):
            v = acc[pl.ds(i * _L, _L)]
            acc[pl.ds(i * _L, _L)] = jnp.where(v > -jnp.inf, v, 0.0)
            return carry

        lax.fori_loop(0, accw // _L, fix_body, 0)
        pltpu.sync_copy(acc, out_hbm.at[pl.ds(wid * accw, accw)])

    f = pl.kernel(
        body,
        out_type=jax.ShapeDtypeStruct((npad * d,), jnp.float32),
        mesh=plsc.VectorSubcoreMesh(**_MESH),
        scratch_types=[
            pltpu.VMEM((accw,), jnp.float32),
            pltpu.VMEM((scan,), jnp.int32),
            pltpu.VMEM((scan + gc,), jnp.int32),
            pltpu.VMEM((scan + gc,), jnp.int32),
            pltpu.VMEM((gc, d), jnp.float32),
            pltpu.VMEM((scan,), jnp.int32),
            pltpu.VMEM((scan + gc,), jnp.int32),
            pltpu.VMEM((scan + gc,), jnp.int32),
            pltpu.VMEM((gc, d), jnp.float32),
            pltpu.SemaphoreType.DMA,
            pltpu.SemaphoreType.DMA,
            pltpu.SemaphoreType.DMA,
            pltpu.SemaphoreType.DMA,
        ],
        compiler_params=pltpu.CompilerParams(needs_layout_passes=False),
    )
    return f(dst, H).reshape(npad, d)[:n]


def kernel(x, edge_index, W1, b1, W2, b2):
    n, d = x.shape
    src = edge_index[0]
    dst = edge_index[1]
    A, B = _node_proj(x, W1, b1)
    G = _edge_gather(A, B, src, dst)
    H = _edge_mlp(G, W2, b2)
    return _segment_max(dst, H, n)


# speedup vs baseline: 1.0034x; 1.0034x over previous
import functools

import jax
import jax.numpy as jnp
from jax import lax
from jax.experimental import pallas as pl
from jax.experimental.pallas import tpu as pltpu
from jax.experimental.pallas import tpu_sc as plsc

_NC = 2
_NS = 16
_NW = _NC * _NS
_L = 16

_MESH = dict(core_axis_name="c", subcore_axis_name="s", num_cores=_NC,
             num_subcores=_NS)


def _proj_body(x_ref, w1_ref, b1_ref, a_ref, b_ref):
    d = x_ref.shape[1]
    x = x_ref[...]
    w1a = w1_ref[:d, :]
    w1b = w1_ref[d:, :]
    a_ref[...] = (
        jnp.dot(x, w1a - w1b, preferred_element_type=jnp.float32) + b1_ref[...]
    )
    b_ref[...] = jnp.dot(x, w1b, preferred_element_type=jnp.float32)


def _node_proj(x, W1, b1):
    n, d = x.shape
    bn = 1000
    return pl.pallas_call(
        _proj_body,
        grid=(n // bn,),
        in_specs=[
            pl.BlockSpec((bn, d), lambda i: (i, 0)),
            pl.BlockSpec((2 * d, d), lambda i: (0, 0)),
            pl.BlockSpec((1, d), lambda i: (0, 0)),
        ],
        out_specs=[
            pl.BlockSpec((bn, d), lambda i: (i, 0)),
            pl.BlockSpec((bn, d), lambda i: (i, 0)),
        ],
        out_shape=[
            jax.ShapeDtypeStruct((n, d), jnp.float32),
            jax.ShapeDtypeStruct((n, d), jnp.float32),
        ],
    )(x, W1, b1.reshape(1, d))


def _edge_gather(A, B, src, dst):
    n, d = A.shape
    e = src.shape[0]
    c2 = 80
    nchunk = e // c2
    cpt = -(-nchunk // _NW)
    npair = -(-cpt // 2)

    def body(a_hbm, b_hbm, src_hbm, dst_hbm, g_hbm,
             idxd0, idxs0, bufa0, bufb0, idxd1, idxs1, bufa1, bufb1,
             semA0, semB0, semO0, semA1, semB1, semO1):
        wid = lax.axis_index("s") * _NC + lax.axis_index("c")
        slots = ((idxd0, idxs0, bufa0, bufb0, semA0, semB0, semO0),
                 (idxd1, idxs1, bufa1, bufb1, semA1, semB1, semO1))

        def pair_body(bi, carry):
            c0 = (2 * bi) * _NW + wid
            cs = (c0, c0 + _NW)
            for s in range(2):
                idxd, idxs, bufa, bufb, semA, semB, semO = slots[s]

                @pl.when(cs[s] < nchunk)
                def _():
                    base = cs[s] * c2
                    pltpu.sync_copy(dst_hbm.at[pl.ds(base, c2)], idxd)
                    pltpu.sync_copy(src_hbm.at[pl.ds(base, c2)], idxs)
                    pltpu.async_copy(a_hbm.at[idxd], bufa, semA)
                    pltpu.async_copy(b_hbm.at[idxs], bufb, semB)

            for s in range(2):
                idxd, idxs, bufa, bufb, semA, semB, semO = slots[s]

                @pl.when(cs[s] < nchunk)
                def _():
                    pltpu.make_async_copy(a_hbm.at[idxd], bufa, semA).wait()
                    pltpu.make_async_copy(b_hbm.at[idxs], bufb, semB).wait()

                    def add_row(r, carry2):
                        for k in range(d // _L):
                            bufa[r, pl.ds(k * _L, _L)] = (
                                bufa[r, pl.ds(k * _L, _L)]
                                + bufb[r, pl.ds(k * _L, _L)])
                        return carry2

                    lax.fori_loop(0, c2, add_row, 0)
                    base = cs[s] * c2
                    pltpu.async_copy(bufa, g_hbm.at[pl.ds(base, c2)], semO)

            for s in range(2):
                idxd, idxs, bufa, bufb, semA, semB, semO = slots[s]

                @pl.when(cs[s] < nchunk)
                def _():
                    base = cs[s] * c2
                    pltpu.make_async_copy(
                        bufa, g_hbm.at[pl.ds(base, c2)], semO).wait()

            return carry

        lax.fori_loop(0, npair, pair_body, 0)

    f = pl.kernel(
        body,
        out_type=jax.ShapeDtypeStruct((e, d), jnp.float32),
        mesh=plsc.VectorSubcoreMesh(**_MESH),
        scratch_types=[
            pltpu.VMEM((c2,), jnp.int32),
            pltpu.VMEM((c2,), jnp.int32),
            pltpu.VMEM((c2, d), jnp.float32),
            pltpu.VMEM((c2, d), jnp.float32),
            pltpu.VMEM((c2,), jnp.int32),
            pltpu.VMEM((c2,), jnp.int32),
            pltpu.VMEM((c2, d), jnp.float32),
            pltpu.VMEM((c2, d), jnp.float32),
            pltpu.SemaphoreType.DMA,
            pltpu.SemaphoreType.DMA,
            pltpu.SemaphoreType.DMA,
            pltpu.SemaphoreType.DMA,
            pltpu.SemaphoreType.DMA,
            pltpu.SemaphoreType.DMA,
        ],
        compiler_params=pltpu.CompilerParams(needs_layout_passes=False),
    )
    return f(A, B, src, dst)


def _mlp_body(g_ref, w2_ref, b2_ref, h_ref):
    g = jnp.maximum(g_ref[...], 0.0)
    h_ref[...] = (
        jnp.dot(g, w2_ref[...], preferred_element_type=jnp.float32) + b2_ref[...]
    )


def _edge_mlp(G, W2, b2):
    e, d = G.shape
    be = 640
    return pl.pallas_call(
        _mlp_body,
        grid=(e // be,),
        in_specs=[
            pl.BlockSpec((be, d), lambda i: (i, 0)),
            pl.BlockSpec((d, d), lambda i: (0, 0)),
            pl.BlockSpec((1, d), lambda i: (0, 0)),
        ],
        out_specs=pl.BlockSpec((be, d), lambda i: (i, 0)),
        out_shape=jax.ShapeDtypeStruct((e, d), jnp.float32),
    )(G, W2, b2.reshape(1, d))


def _segment_max(dst, H, n):
    e, d = H.shape
    npt = -(-n // _NW)
    npad = npt * _NW
    scan = 2000
    nscan = e // scan
    gc = 64
    accw = npt * d

    assert nscan % 2 == 0
    npairs = nscan // 2

    def body(dst_hbm, h_hbm, out_hbm, acc,
             dbufA, ebufA, lbufA, hbufA, dbufB, ebufB, lbufB, hbufB,
             semDA, semHA, semDB, semHB):
        wid = lax.axis_index("s") * _NC + lax.axis_index("c")
        lo = wid * npt
        neg_inf = jnp.full((_L,), -jnp.inf, jnp.float32)
        iota = lax.iota(jnp.int32, _L)

        def init_body(i, carry):
            acc[pl.ds(i * _L, _L)] = neg_inf
            return carry

        lax.fori_loop(0, accw // _L, init_body, 0)

        def scan_chunk(sc, dbuf, ebuf, lbuf):
            def filt_body(k, cnt):
                dv = dbuf[pl.ds(k * _L, _L)]
                m = (dv >= lo) & (dv < lo + npt)
                mi = jnp.where(m, 1, 0)
                csum = plsc.cumsum(mi)
                pos = cnt + csum - mi
                eid = sc * scan + k * _L + iota
                plsc.store_scatter(ebuf, [pos], eid, mask=m)
                plsc.store_scatter(lbuf, [pos], dv - lo, mask=m)
                return cnt + plsc.all_reduce_population_count(m)

            cnt0 = jnp.zeros((_L,), jnp.int32)
            cvec = lax.fori_loop(0, scan // _L, filt_body, cnt0)
            zeros = jnp.zeros((_L,), jnp.int32)
            dump = jnp.full((_L,), npt, jnp.int32)
            for t in range(gc // _L):
                plsc.store_scatter(ebuf, [cvec + t * _L + iota], zeros)
                plsc.store_scatter(lbuf, [cvec + t * _L + iota], dump)
            return cvec[0]

        def accum_group(g, lbuf, hbuf):
            def blk_body(q, carry2):
                dlv = lbuf[pl.ds(g * gc + q * _L, _L)] * d
                rbase = q * _L
                for j in range(_L):
                    rb = dlv[j]
                    avs = [acc[pl.ds(rb + k * _L, _L)]
                           for k in range(d // _L)]
                    hvs = [hbuf[rbase + j, pl.ds(k * _L, _L)]
                           for k in range(d // _L)]
                    for k in range(d // _L):
                        acc[pl.ds(rb + k * _L, _L)] = jnp.maximum(
                            avs[k], hvs[k])
                return carry2

            lax.fori_loop(0, gc // _L, blk_body, 0)

        def extra_groups(count, ebuf, lbuf, hbuf, semH):
            ngroups = (count + gc - 1) // gc

            def g_body(g, carry2):
                pltpu.async_copy(h_hbm.at[ebuf.at[pl.ds(g * gc, gc)]],
                                 hbuf, semH).wait()
                accum_group(g, lbuf, hbuf)
                return carry2

            lax.fori_loop(1, ngroups, g_body, 0)

        pltpu.async_copy(dst_hbm.at[pl.ds(0, scan)], dbufA, semDA)

        def pair_body(bi, carry):
            sc0 = 2 * bi
            sc1 = sc0 + 1
            pltpu.make_async_copy(
                dst_hbm.at[pl.ds(sc0 * scan, scan)], dbufA, semDA).wait()
            pltpu.async_copy(dst_hbm.at[pl.ds(sc1 * scan, scan)], dbufB, semDB)
            count0 = scan_chunk(sc0, dbufA, ebufA, lbufA)
            pltpu.async_copy(h_hbm.at[ebufA.at[pl.ds(0, gc)]], hbufA, semHA)

            @pl.when(bi + 1 < npairs)
            def _():
                pltpu.async_copy(
                    dst_hbm.at[pl.ds((sc0 + 2) * scan, scan)], dbufA, semDA)

            pltpu.make_async_copy(
                dst_hbm.at[pl.ds(sc1 * scan, scan)], dbufB, semDB).wait()
            count1 = scan_chunk(sc1, dbufB, ebufB, lbufB)
            pltpu.async_copy(h_hbm.at[ebufB.at[pl.ds(0, gc)]], hbufB, semHB)

            pltpu.make_async_copy(
                h_hbm.at[ebufA.at[pl.ds(0, gc)]], hbufA, semHA).wait()
            accum_group(0, lbufA, hbufA)
            extra_groups(count0, ebufA, lbufA, hbufA, semHA)

            pltpu.make_async_copy(
                h_hbm.at[ebufB.at[pl.ds(0, gc)]], hbufB, semHB).wait()
            accum_group(0, lbufB, hbufB)
            extra_groups(count1, ebufB, lbufB, hbufB, semHB)
            return carry

        lax.fori_loop(0, npairs, pair_body, 0)

        def fix_body(i, carry):
            v = acc[pl.ds(i * _L, _L)]
            acc[pl.ds(i * _L, _L)] = jnp.where(v > -jnp.inf, v, 0.0)
            return carry

        lax.fori_loop(0, accw // _L, fix_body, 0)
        pltpu.sync_copy(acc.at[pl.ds(0, accw)],
                        out_hbm.at[pl.ds(wid * accw, accw)])

    f = pl.kernel(
        body,
        out_type=jax.ShapeDtypeStruct((npad * d,), jnp.float32),
        mesh=plsc.VectorSubcoreMesh(**_MESH),
        scratch_types=[
            pltpu.VMEM((accw + d,), jnp.float32),
            pltpu.VMEM((scan,), jnp.int32),
            pltpu.VMEM((scan + gc,), jnp.int32),
            pltpu.VMEM((scan + gc,), jnp.int32),
            pltpu.VMEM((gc, d), jnp.float32),
            pltpu.VMEM((scan,), jnp.int32),
            pltpu.VMEM((scan + gc,), jnp.int32),
            pltpu.VMEM((scan + gc,), jnp.int32),
            pltpu.VMEM((gc, d), jnp.float32),
            pltpu.SemaphoreType.DMA,
            pltpu.SemaphoreType.DMA,
            pltpu.SemaphoreType.DMA,
            pltpu.SemaphoreType.DMA,
        ],
        compiler_params=pltpu.CompilerParams(needs_layout_passes=False),
    )
    return f(dst, H).reshape(npad, d)[:n]


def kernel(x, edge_index, W1, b1, W2, b2):
    n, d = x.shape
    src = edge_index[0]
    dst = edge_index[1]
    A, B = _node_proj(x, W1, b1)
    G = _edge_gather(A, B, src, dst)
    H = _edge_mlp(G, W2, b2)
    return _segment_max(dst, H, n)
